# baseline (speedup 1.0000x reference)
import functools

import jax
import jax.numpy as jnp
from jax import lax
from jax.experimental import pallas as pl
from jax.experimental.pallas import tpu as pltpu
from jax.experimental.pallas import tpu_sc as plsc

_VOCAB = 1000
_BATCH = 4096
_HIST = 20
_D = _VOCAB
_DP = 1024
_HP = 24

_NC = 2
_NS = 16
_NW = _NC * _NS
_SPW = _BATCH // _NW

_mesh = plsc.VectorSubcoreMesh(
    core_axis_name="c", subcore_axis_name="s", num_cores=_NC, num_subcores=_NS
)


@functools.partial(
    pl.kernel,
    out_type=jax.ShapeDtypeStruct((_BATCH, _HIST, _D), jnp.float32),
    mesh=_mesh,
    scratch_types=[
        pltpu.VMEM((_SPW * _HP,), jnp.int32),
        pltpu.VMEM((_HP, 8, 128), jnp.float32),
        pltpu.VMEM((_HP, 8, 128), jnp.float32),
        pltpu.VMEM((_HIST, _D), jnp.float32),
        pltpu.VMEM((_HIST, _D), jnp.float32),
        pltpu.SemaphoreType.DMA,
        pltpu.SemaphoreType.DMA,
        pltpu.SemaphoreType.DMA,
        pltpu.SemaphoreType.DMA,
    ],
)
def _embed_lookup(
    idx_hbm, table_hbm, out_hbm,
    idx_v, bufp0, bufp1, buf0, buf1, gsem0, gsem1, wsem0, wsem1,
):
    wid = lax.axis_index("s") * _NC + lax.axis_index("c")
    base = wid * _SPW
    pltpu.sync_copy(idx_hbm.at[pl.ds(base * _HP, _SPW * _HP)], idx_v)

    def gather_start(s, bufp, sem):
        idx_slice = idx_v.at[pl.ds(s * _HP, _HP)]
        pltpu.async_copy(table_hbm.at[idx_slice], bufp, sem)

    def gather_wait(s, bufp, sem):
        idx_slice = idx_v.at[pl.ds(s * _HP, _HP)]
        pltpu.make_async_copy(table_hbm.at[idx_slice], bufp, sem).wait()

    def write_start(s, buf, sem):
        pltpu.async_copy(buf, out_hbm.at[base + s], sem)

    def write_wait(s, buf, sem):
        pltpu.make_async_copy(buf, out_hbm.at[base + s], sem).wait()

    def trim(bufp, buf):
        def row(r, carry):
            for k in range(62):
                c = 16 * k
                buf[r, pl.ds(c, 16)] = bufp[r, c // 128, pl.ds(c % 128, 16)]
            tail = pl.multiple_of(jnp.int32(992), 16)
            buf[r, pl.ds(tail, 16)] = bufp[r, 7, pl.ds(96, 16)]
            return carry
        lax.fori_loop(0, _HIST, row, 0)

    gather_start(0, bufp0, gsem0)
    gather_start(1, bufp1, gsem1)
    gather_wait(0, bufp0, gsem0)
    trim(bufp0, buf0)
    gather_start(2, bufp0, gsem0)
    write_start(0, buf0, wsem0)
    gather_wait(1, bufp1, gsem1)
    trim(bufp1, buf1)
    gather_start(3, bufp1, gsem1)
    write_start(1, buf1, wsem1)

    def body(t, carry):
        s0 = 2 * t
        gather_wait(s0, bufp0, gsem0)
        write_wait(s0 - 2, buf0, wsem0)
        trim(bufp0, buf0)
        gather_start(s0 + 2, bufp0, gsem0)
        write_start(s0, buf0, wsem0)
        gather_wait(s0 + 1, bufp1, gsem1)
        write_wait(s0 - 1, buf1, wsem1)
        trim(bufp1, buf1)
        gather_start(s0 + 3, bufp1, gsem1)
        write_start(s0 + 1, buf1, wsem1)
        return carry

    lax.fori_loop(1, _SPW // 2 - 1, body, 0)

    s0 = _SPW - 2
    gather_wait(s0, bufp0, gsem0)
    write_wait(s0 - 2, buf0, wsem0)
    trim(bufp0, buf0)
    write_start(s0, buf0, wsem0)
    gather_wait(s0 + 1, bufp1, gsem1)
    write_wait(s0 - 1, buf1, wsem1)
    trim(bufp1, buf1)
    write_start(s0 + 1, buf1, wsem1)
    write_wait(s0, buf0, wsem0)
    write_wait(s0 + 1, buf1, wsem1)


def kernel(x, token_embedding_weight):
    idx = jnp.pad(x.astype(jnp.int32), ((0, 0), (0, _HP - _HIST))).reshape(-1)
    table = jnp.pad(token_embedding_weight, ((0, 0), (0, _DP - _D)))
    table = table.reshape(_VOCAB, 8, 128)
    return _embed_lookup(idx, table)


# speedup vs baseline: 2.0144x; 2.0144x over previous
import functools

import jax
import jax.numpy as jnp
from jax import lax
from jax.experimental import pallas as pl
from jax.experimental.pallas import tpu as pltpu
from jax.experimental.pallas import tpu_sc as plsc

_VOCAB = 1000
_BATCH = 4096
_HIST = 20
_D = _VOCAB
_B = _BATCH * _HIST

_NC = 2
_NS = 16
_NW = _NC * _NS
_BPW = _B // _NW
_C = 64
_NCHUNK = _BPW // _C

_mesh = plsc.VectorSubcoreMesh(
    core_axis_name="c", subcore_axis_name="s", num_cores=_NC, num_subcores=_NS
)


@functools.partial(
    pl.kernel,
    out_type=jax.ShapeDtypeStruct((_B, _D), jnp.float32),
    mesh=_mesh,
    scratch_types=[
        pltpu.VMEM((_BPW,), jnp.int32),
        pltpu.VMEM((_C, _D), jnp.float32),
        pltpu.VMEM((_C, _D), jnp.float32),
        pltpu.SemaphoreType.DMA,
        pltpu.SemaphoreType.DMA,
    ],
    compiler_params=pltpu.CompilerParams(use_tc_tiling_on_sc=False),
)
def _embed_lookup(idx_hbm, table_hbm, out_hbm, idx_v, buf0, buf1, sem0, sem1):
    wid = lax.axis_index("s") * _NC + lax.axis_index("c")
    base = wid * _BPW
    pltpu.sync_copy(idx_hbm.at[pl.ds(base, _BPW)], idx_v)

    def gather_start(chunk, buf, sem):
        idx_slice = idx_v.at[pl.ds(chunk * _C, _C)]
        pltpu.async_copy(table_hbm.at[idx_slice], buf, sem)

    def gather_wait(chunk, buf, sem):
        idx_slice = idx_v.at[pl.ds(chunk * _C, _C)]
        pltpu.make_async_copy(table_hbm.at[idx_slice], buf, sem).wait()

    def write_out(chunk, buf):
        pltpu.sync_copy(buf, out_hbm.at[pl.ds(base + chunk * _C, _C)])

    gather_start(0, buf0, sem0)
    gather_start(1, buf1, sem1)

    def body(t, carry):
        c0 = 2 * t
        gather_wait(c0, buf0, sem0)
        write_out(c0, buf0)
        gather_start(c0 + 2, buf0, sem0)
        gather_wait(c0 + 1, buf1, sem1)
        write_out(c0 + 1, buf1)
        gather_start(c0 + 3, buf1, sem1)
        return carry

    lax.fori_loop(0, _NCHUNK // 2 - 1, body, 0)

    gather_wait(_NCHUNK - 2, buf0, sem0)
    write_out(_NCHUNK - 2, buf0)
    gather_wait(_NCHUNK - 1, buf1, sem1)
    write_out(_NCHUNK - 1, buf1)


def kernel(x, token_embedding_weight):
    idx = x.reshape(-1).astype(jnp.int32)
    out = _embed_lookup(idx, token_embedding_weight)
    return out.reshape(_BATCH, _HIST, _VOCAB)
